# Initial kernel scaffold; baseline (speedup 1.0000x reference)
import jax
import jax.numpy as jnp
from jax.experimental import pallas as pl


def kernel(x, edge_index, norm_elev, norm_length, norm_geom_1, norm_in_offset, norm_out_offset, W1, b1, W2, b2):
    raise NotImplementedError("write your pallas kernel here")


# speedup vs baseline: 71.5612x; 71.5612x over previous
import functools

import jax
import jax.numpy as jnp
from jax import lax
from jax.experimental import pallas as pl
from jax.experimental.pallas import tpu as pltpu
from jax.experimental.pallas import tpu_sc as plsc

NSUB = 16


def _body(np_, e, chunk,
          tab0, outz, srcg, dstg, ino, outo, geom, leng, wpack,
          out_hbm,
          tab_sp, out_sp,
          srcb, dstb, inob, outob, geomb, lengb,
          rows_d, rows_s, mrow, mbuf, wbuf):
  wid = lax.axis_index("s")
  npt = np_ // NSUB
  ept = e // NSUB
  nch = ept // chunk
  grp = chunk // 16

  iota16 = lax.iota(jnp.int32, 16)
  cols = [jnp.full((16,), c, jnp.int32) for c in range(8)]
  zf = jnp.zeros((16,), jnp.float32)

  done = 0
  while done < npt:
    piece = min(chunk, npt - done)
    rsl = pl.ds(wid * npt + done, piece)
    pltpu.sync_copy(tab0.at[rsl], rows_d.at[pl.ds(0, piece)])
    pltpu.sync_copy(rows_d.at[pl.ds(0, piece)], tab_sp.at[rsl])
    pltpu.sync_copy(outz.at[pl.ds(wid * npt + done, piece)],
                    mbuf.at[pl.ds(0, piece)])
    pltpu.sync_copy(mbuf.at[pl.ds(0, piece)],
                    out_sp.at[pl.ds(wid * npt + done, piece)])
    done += piece
  pltpu.sync_copy(wpack, wbuf)

  def zero_mrow(j, c):
    ids = j * 16 + iota16
    for col in (0, 1, 2, 6, 7):
      plsc.store_scatter(mrow, [ids, cols[col]], zf)
    return c
  lax.fori_loop(0, grp, zero_mrow, 0)

  def _splat(i):
    return plsc.load_gather(wbuf, [jnp.full((16,), i, jnp.int32)])

  plsc.subcore_barrier()

  w1 = [[_splat(1 + k * 3 + j) for j in range(3)] for k in range(6)]
  b1 = [_splat(19 + j) for j in range(3)]

  def passA(c, carry):
    esl = pl.ds(wid * ept + c * chunk, chunk)
    pltpu.sync_copy(srcg.at[esl], srcb)
    pltpu.sync_copy(dstg.at[esl], dstb)
    pltpu.sync_copy(ino.at[esl], inob)
    pltpu.sync_copy(outo.at[esl], outob)
    pltpu.sync_copy(geom.at[esl], geomb)
    pltpu.sync_copy(leng.at[esl], lengb)
    pltpu.sync_copy(tab_sp.at[dstb], rows_d)
    pltpu.sync_copy(tab_sp.at[srcb], rows_s)

    def grp_body(j, cc):
      ids = j * 16 + iota16
      sl = pl.ds(j * 16, 16)
      xd0 = plsc.load_gather(rows_d, [ids, cols[0]])
      xd1 = plsc.load_gather(rows_d, [ids, cols[1]])
      ed = plsc.load_gather(rows_d, [ids, cols[2]])
      xs0 = plsc.load_gather(rows_s, [ids, cols[0]])
      xs1 = plsc.load_gather(rows_s, [ids, cols[1]])
      es = plsc.load_gather(rows_s, [ids, cols[2]])
      e1 = (es + outob[sl]) - (ed + inob[sl])
      wv = 1.0 / (lengb[sl] + 1.0)
      feats = (xd0, xd1, xs0, xs1, e1, geomb[sl])
      for jc in range(3):
        acc = b1[jc]
        for k in range(6):
          acc = acc + feats[k] * w1[k][jc]
        acc = jnp.maximum(acc, 0.0) * wv
        plsc.store_scatter(mrow, [ids, cols[3 + jc]], acc)
      return cc

    lax.fori_loop(0, grp, grp_body, 0)
    pltpu.sync_copy(mrow, tab_sp.at[dstb], add=True)
    return carry

  lax.fori_loop(0, nch, passA, 0)
  plsc.subcore_barrier()

  w2 = [_splat(22 + k) for k in range(8)]
  b2 = _splat(30)

  def passB(c, carry):
    esl = pl.ds(wid * ept + c * chunk, chunk)
    pltpu.sync_copy(srcg.at[esl], srcb)
    pltpu.sync_copy(dstg.at[esl], dstb)
    pltpu.sync_copy(ino.at[esl], inob)
    pltpu.sync_copy(outo.at[esl], outob)
    pltpu.sync_copy(geom.at[esl], geomb)
    pltpu.sync_copy(leng.at[esl], lengb)
    pltpu.sync_copy(tab_sp.at[dstb], rows_d)
    pltpu.sync_copy(tab_sp.at[srcb], rows_s)

    def grp_body(j, cc):
      ids = j * 16 + iota16
      sl = pl.ds(j * 16, 16)
      hd0 = plsc.load_gather(rows_d, [ids, cols[3]])
      hd1 = plsc.load_gather(rows_d, [ids, cols[4]])
      hd2 = plsc.load_gather(rows_d, [ids, cols[5]])
      ed = plsc.load_gather(rows_d, [ids, cols[2]])
      hs0 = plsc.load_gather(rows_s, [ids, cols[3]])
      hs1 = plsc.load_gather(rows_s, [ids, cols[4]])
      hs2 = plsc.load_gather(rows_s, [ids, cols[5]])
      es = plsc.load_gather(rows_s, [ids, cols[2]])
      e1 = (es + outob[sl]) - (ed + inob[sl])
      wv = 1.0 / (lengb[sl] + 1.0)
      feats = (hd0, hd1, hd2, hs0, hs1, hs2, e1, geomb[sl])
      acc = b2
      for k in range(8):
        acc = acc + feats[k] * w2[k]
      mbuf[sl] = jnp.maximum(acc, 0.0) * wv
      return cc

    lax.fori_loop(0, grp, grp_body, 0)
    pltpu.sync_copy(mbuf.at[pl.ds(0, chunk)], out_sp.at[dstb], add=True)
    return carry

  lax.fori_loop(0, nch, passB, 0)
  plsc.subcore_barrier()

  done = 0
  while done < npt:
    piece = min(chunk, npt - done)
    rsl = pl.ds(wid * npt + done, piece)
    pltpu.sync_copy(out_sp.at[rsl], mbuf.at[pl.ds(0, piece)])
    pltpu.sync_copy(mbuf.at[pl.ds(0, piece)], out_hbm.at[rsl])
    done += piece


@functools.lru_cache(maxsize=None)
def _build(np_, e, chunk):
  mesh = plsc.VectorSubcoreMesh(
      core_axis_name="c", subcore_axis_name="s", num_cores=1,
      num_subcores=NSUB)
  mx = max(chunk, np_ // NSUB)
  return pl.kernel(
      functools.partial(_body, np_, e, chunk),
      out_type=jax.ShapeDtypeStruct((np_,), jnp.float32),
      mesh=mesh,
      scratch_types=[
          pltpu.VMEM_SHARED((np_, 8), jnp.float32),
          pltpu.VMEM_SHARED((np_,), jnp.float32),
          pltpu.VMEM((chunk,), jnp.int32),
          pltpu.VMEM((chunk,), jnp.int32),
          pltpu.VMEM((chunk,), jnp.float32),
          pltpu.VMEM((chunk,), jnp.float32),
          pltpu.VMEM((chunk,), jnp.float32),
          pltpu.VMEM((chunk,), jnp.float32),
          pltpu.VMEM((chunk, 8), jnp.float32),
          pltpu.VMEM((chunk, 8), jnp.float32),
          pltpu.VMEM((chunk, 8), jnp.float32),
          pltpu.VMEM((mx,), jnp.float32),
          pltpu.VMEM((32,), jnp.float32),
      ],
      compiler_params=pltpu.CompilerParams(
          needs_layout_passes=False, use_tc_tiling_on_sc=False),
  )


def kernel(x, edge_index, norm_elev, norm_length, norm_geom_1,
           norm_in_offset, norm_out_offset, W1, b1, W2, b2,
           _chunk=2000):
  n = x.shape[0]
  e = edge_index.shape[1]
  np_ = ((n + 8 * NSUB - 1) // (8 * NSUB)) * (8 * NSUB)
  padn = np_ - n
  elev = jnp.pad(norm_elev, (0, padn))
  tab0 = jnp.concatenate(
      [jnp.pad(x, ((0, padn), (0, 0))), elev[:, None],
       jnp.zeros((np_, 5), jnp.float32)], axis=1)
  outz = jnp.zeros((np_,), jnp.float32)
  wpack = jnp.concatenate(
      [jnp.zeros((1,), jnp.float32), W1.reshape(-1), b1, W2.reshape(-1), b2,
       jnp.zeros((1,), jnp.float32)])
  out = _build(np_, e, _chunk)(
      tab0, outz, edge_index[0], edge_index[1],
      norm_in_offset, norm_out_offset, norm_geom_1, norm_length, wpack)
  return out[:n, None]


# speedup vs baseline: 126.9828x; 1.7745x over previous
import functools

import jax
import jax.numpy as jnp
from jax import lax
from jax.experimental import pallas as pl
from jax.experimental.pallas import tpu as pltpu
from jax.experimental.pallas import tpu_sc as plsc

NSUB = 16
CHUNK = 800


def _body(np_, e,
          tab0, outz, srcg, dstg, ino, outo, geom, leng, wpack,
          out_hbm,
          tab_sp, out_sp,
          srcb, inob, outob, geomb, lengb,
          dstb, mrow, mbuf,
          rows_d, rows_s,
          wbuf, slin, sgat, ssca):
  chunk = CHUNK
  wid = lax.axis_index("s")
  npt = np_ // NSUB
  ept = e // NSUB
  nch = ept // chunk
  grp = chunk // 16

  iota16 = lax.iota(jnp.int32, 16)
  cols = [jnp.full((16,), c, jnp.int32) for c in range(8)]
  zf = jnp.zeros((16,), jnp.float32)

  done = 0
  while done < npt:
    piece = min(chunk, npt - done)
    rsl = pl.ds(wid * npt + done, piece)
    pltpu.sync_copy(tab0.at[rsl], rows_d[0].at[pl.ds(0, piece)])
    pltpu.sync_copy(rows_d[0].at[pl.ds(0, piece)], tab_sp.at[rsl])
    pltpu.sync_copy(outz.at[pl.ds(wid * npt + done, piece)],
                    mbuf[0].at[pl.ds(0, piece)])
    pltpu.sync_copy(mbuf[0].at[pl.ds(0, piece)],
                    out_sp.at[pl.ds(wid * npt + done, piece)])
    done += piece
  pltpu.sync_copy(wpack, wbuf)

  def zero_mrow(j, c):
    ids = j * 16 + iota16
    for b in range(3):
      for col in (0, 1, 2, 6, 7):
        plsc.store_scatter(mrow[b], [ids, cols[col]], zf)
    return c
  lax.fori_loop(0, grp, zero_mrow, 0)

  def _splat(i):
    return plsc.load_gather(wbuf, [jnp.full((16,), i, jnp.int32)])

  plsc.subcore_barrier()

  w1 = [[_splat(1 + k * 3 + j) for j in range(3)] for k in range(6)]
  b1 = [_splat(19 + j) for j in range(3)]
  w2 = [_splat(22 + k) for k in range(8)]
  b2 = _splat(30)

  def esl(c):
    return pl.ds(wid * ept + c * chunk, chunk)

  def lin_pairs(c, b2, b3):
    return [(srcg.at[esl(c)], srcb[b2]),
            (dstg.at[esl(c)], dstb[b3]),
            (ino.at[esl(c)], inob[b2]),
            (outo.at[esl(c)], outob[b2]),
            (geom.at[esl(c)], geomb[b2]),
            (leng.at[esl(c)], lengb[b2])]

  def issue_lin(c, b2, b3):
    for s, d in lin_pairs(c, b2, b3):
      pltpu.async_copy(s, d, slin[b2])

  def wait_lin(c, b2, b3):
    for s, d in lin_pairs(c, b2, b3):
      pltpu.make_async_copy(s, d, slin[b2]).wait()

  def gat_pairs(b2, b3):
    return [(tab_sp.at[dstb[b3]], rows_d[b2]),
            (tab_sp.at[srcb[b2]], rows_s[b2])]

  def issue_gat(b2, b3):
    for s, d in gat_pairs(b2, b3):
      pltpu.async_copy(s, d, sgat[b2])

  def wait_gat(b2, b3):
    for s, d in gat_pairs(b2, b3):
      pltpu.make_async_copy(s, d, sgat[b2]).wait()

  def computeA(pb2, pb3):
    rd, rs = rows_d[pb2], rows_s[pb2]

    def grp_body(j, cc):
      ids = j * 16 + iota16
      sl = pl.ds(j * 16, 16)
      xd0 = plsc.load_gather(rd, [ids, cols[0]])
      xd1 = plsc.load_gather(rd, [ids, cols[1]])
      ed = plsc.load_gather(rd, [ids, cols[2]])
      xs0 = plsc.load_gather(rs, [ids, cols[0]])
      xs1 = plsc.load_gather(rs, [ids, cols[1]])
      es = plsc.load_gather(rs, [ids, cols[2]])
      e1 = (es + outob[pb2][sl]) - (ed + inob[pb2][sl])
      wv = 1.0 / (lengb[pb2][sl] + 1.0)
      feats = (xd0, xd1, xs0, xs1, e1, geomb[pb2][sl])
      for jc in range(3):
        acc = b1[jc]
        for k in range(6):
          acc = acc + feats[k] * w1[k][jc]
        acc = jnp.maximum(acc, 0.0) * wv
        plsc.store_scatter(mrow[pb3], [ids, cols[3 + jc]], acc)
      return cc

    lax.fori_loop(0, grp, grp_body, 0)

  def sca_pairA(pb3):
    return (mrow[pb3], tab_sp.at[dstb[pb3]])

  def computeB(pb2, pb3):
    rd, rs = rows_d[pb2], rows_s[pb2]

    def grp_body(j, cc):
      ids = j * 16 + iota16
      sl = pl.ds(j * 16, 16)
      hd0 = plsc.load_gather(rd, [ids, cols[3]])
      hd1 = plsc.load_gather(rd, [ids, cols[4]])
      hd2 = plsc.load_gather(rd, [ids, cols[5]])
      ed = plsc.load_gather(rd, [ids, cols[2]])
      hs0 = plsc.load_gather(rs, [ids, cols[3]])
      hs1 = plsc.load_gather(rs, [ids, cols[4]])
      hs2 = plsc.load_gather(rs, [ids, cols[5]])
      es = plsc.load_gather(rs, [ids, cols[2]])
      e1 = (es + outob[pb2][sl]) - (ed + inob[pb2][sl])
      wv = 1.0 / (lengb[pb2][sl] + 1.0)
      feats = (hd0, hd1, hd2, hs0, hs1, hs2, e1, geomb[pb2][sl])
      acc = b2
      for k in range(8):
        acc = acc + feats[k] * w2[k]
      mbuf[pb3][sl] = jnp.maximum(acc, 0.0) * wv
      return cc

    lax.fori_loop(0, grp, grp_body, 0)

  def run_pass(compute, sca_pair):
    def issue_sca(b3):
      s, d = sca_pair(b3)
      pltpu.async_copy(s, d, ssca[b3], add=True)

    def wait_sca(b3):
      s, d = sca_pair(b3)
      pltpu.make_async_copy(s, d, ssca[b3]).wait()

    def iteration(c, u, first=False, issue_g=True, issue_l=True):
      b2, b3 = u % 2, u % 3
      nb2, nb3 = (u + 1) % 2, (u + 1) % 3
      wait_gat(b2, b3)
      compute(b2, b3)
      issue_sca(b3)
      if issue_g:
        wait_lin(c + 1, nb2, nb3)
        issue_gat(nb2, nb3)
      if not first:
        wait_sca((u + 2) % 3)
      if issue_l:
        issue_lin(c + 2, b2, (u + 2) % 3)

    issue_lin(0, 0, 0)
    issue_lin(1, 1, 1)
    wait_lin(0, 0, 0)
    issue_gat(0, 0)
    iteration(0, 0, first=True)
    iteration(1, 1)
    nsteady = nch - 4
    assert nsteady % 6 == 0

    def steady(k6, cc):
      for u in range(6):
        iteration(2 + k6 * 6 + u, 2 + u)
      return cc

    lax.fori_loop(0, nsteady // 6, steady, 0)
    iteration(nch - 2, nch - 2, issue_g=True, issue_l=False)
    iteration(nch - 1, nch - 1, issue_g=False, issue_l=False)
    wait_sca((nch - 1) % 3)

  run_pass(computeA, sca_pairA)
  plsc.subcore_barrier()

  def sca_pairB(pb3):
    return (mbuf[pb3], out_sp.at[dstb[pb3]])

  run_pass(computeB, sca_pairB)
  plsc.subcore_barrier()

  done = 0
  while done < npt:
    piece = min(chunk, npt - done)
    rsl = pl.ds(wid * npt + done, piece)
    pltpu.sync_copy(out_sp.at[rsl], mbuf[0].at[pl.ds(0, piece)])
    pltpu.sync_copy(mbuf[0].at[pl.ds(0, piece)], out_hbm.at[rsl])
    done += piece


def _body_wrap(np_, e, tab0, outz, srcg, dstg, ino, outo, geom, leng, wpack,
               out_hbm, tab_sp, out_sp,
               srcb0, srcb1, inob0, inob1, outob0, outob1,
               geomb0, geomb1, lengb0, lengb1,
               dstb0, dstb1, dstb2, mrow0, mrow1, mrow2,
               mbuf0, mbuf1, mbuf2, rows_d0, rows_d1, rows_s0, rows_s1,
               wbuf, slin0, slin1, sgat0, sgat1, ssca0, ssca1, ssca2):
  _body(np_, e, tab0, outz, srcg, dstg, ino, outo, geom, leng, wpack,
        out_hbm, tab_sp, out_sp,
        (srcb0, srcb1), (inob0, inob1), (outob0, outob1),
        (geomb0, geomb1), (lengb0, lengb1),
        (dstb0, dstb1, dstb2), (mrow0, mrow1, mrow2),
        (mbuf0, mbuf1, mbuf2), (rows_d0, rows_d1), (rows_s0, rows_s1),
        wbuf, (slin0, slin1), (sgat0, sgat1), (ssca0, ssca1, ssca2))


@functools.lru_cache(maxsize=None)
def _build(np_, e):
  mesh = plsc.VectorSubcoreMesh(
      core_axis_name="c", subcore_axis_name="s", num_cores=1,
      num_subcores=NSUB)
  chunk = CHUNK
  f32, i32 = jnp.float32, jnp.int32
  scratch = [
      pltpu.VMEM_SHARED((np_, 8), f32),
      pltpu.VMEM_SHARED((np_,), f32),
  ]
  scratch += [pltpu.VMEM((chunk,), i32) for _ in range(2)]
  scratch += [pltpu.VMEM((chunk,), f32) for _ in range(8)]
  scratch += [pltpu.VMEM((chunk,), i32) for _ in range(3)]
  scratch += [pltpu.VMEM((chunk, 8), f32) for _ in range(3)]
  scratch += [pltpu.VMEM((chunk,), f32) for _ in range(3)]
  scratch += [pltpu.VMEM((chunk, 8), f32) for _ in range(4)]
  scratch += [pltpu.VMEM((32,), f32)]
  scratch += [pltpu.SemaphoreType.DMA for _ in range(7)]
  return pl.kernel(
      functools.partial(_body_wrap, np_, e),
      out_type=jax.ShapeDtypeStruct((np_,), f32),
      mesh=mesh,
      scratch_types=scratch,
      compiler_params=pltpu.CompilerParams(
          needs_layout_passes=False, use_tc_tiling_on_sc=False),
  )


def kernel(x, edge_index, norm_elev, norm_length, norm_geom_1,
           norm_in_offset, norm_out_offset, W1, b1, W2, b2):
  n = x.shape[0]
  e = edge_index.shape[1]
  np_ = ((n + 8 * NSUB - 1) // (8 * NSUB)) * (8 * NSUB)
  padn = np_ - n
  elev = jnp.pad(norm_elev, (0, padn))
  tab0 = jnp.concatenate(
      [jnp.pad(x, ((0, padn), (0, 0))), elev[:, None],
       jnp.zeros((np_, 5), jnp.float32)], axis=1)
  outz = jnp.zeros((np_,), jnp.float32)
  wpack = jnp.concatenate(
      [jnp.zeros((1,), jnp.float32), W1.reshape(-1), b1, W2.reshape(-1), b2,
       jnp.zeros((1,), jnp.float32)])
  out = _build(np_, e)(
      tab0, outz, edge_index[0], edge_index[1],
      norm_in_offset, norm_out_offset, norm_geom_1, norm_length, wpack)
  return out[:n, None]


# speedup vs baseline: 202.1120x; 1.5916x over previous
import functools

import jax
import jax.numpy as jnp
from jax import lax
from jax.experimental import pallas as pl
from jax.experimental.pallas import tpu as pltpu
from jax.experimental.pallas import tpu_sc as plsc

NSUB = 16
NC = 2
CHUNK = 800


def _pipeline_helpers(ept, chunk, wid, tab_sp, srcg, dstg, ino, outo, geom,
                      leng, srcb, inob, outob, geomb, lengb, dstb,
                      rows_d, rows_s, slin, sgat):
  def esl(c):
    return pl.ds(wid * ept + c * chunk, chunk)

  def lin_pairs(c, b2, b3):
    return [(srcg.at[esl(c)], srcb[b2]),
            (dstg.at[esl(c)], dstb[b3]),
            (ino.at[esl(c)], inob[b2]),
            (outo.at[esl(c)], outob[b2]),
            (geom.at[esl(c)], geomb[b2]),
            (leng.at[esl(c)], lengb[b2])]

  def issue_lin(c, b2, b3):
    for s, d in lin_pairs(c, b2, b3):
      pltpu.async_copy(s, d, slin[b2])

  def wait_lin(c, b2, b3):
    for s, d in lin_pairs(c, b2, b3):
      pltpu.make_async_copy(s, d, slin[b2]).wait()

  def gat_pairs(b2, b3):
    return [(tab_sp.at[dstb[b3]], rows_d[b2]),
            (tab_sp.at[srcb[b2]], rows_s[b2])]

  def issue_gat(b2, b3):
    for s, d in gat_pairs(b2, b3):
      pltpu.async_copy(s, d, sgat[b2])

  def wait_gat(b2, b3):
    for s, d in gat_pairs(b2, b3):
      pltpu.make_async_copy(s, d, sgat[b2]).wait()

  return issue_lin, wait_lin, issue_gat, wait_gat


def _run_pass(nch, compute, sca_pair, ssca,
              issue_lin, wait_lin, issue_gat, wait_gat):
  def issue_sca(b3):
    s, d = sca_pair(b3)
    pltpu.async_copy(s, d, ssca[b3], add=True)

  def wait_sca(b3):
    s, d = sca_pair(b3)
    pltpu.make_async_copy(s, d, ssca[b3]).wait()

  def iteration(c, u, first=False, issue_g=True, issue_l=True):
    b2, b3 = u % 2, u % 3
    nb2, nb3 = (u + 1) % 2, (u + 1) % 3
    wait_gat(b2, b3)
    compute(b2, b3)
    issue_sca(b3)
    if issue_g:
      wait_lin(c + 1, nb2, nb3)
      issue_gat(nb2, nb3)
    if not first:
      wait_sca((u + 2) % 3)
    if issue_l:
      issue_lin(c + 2, b2, (u + 2) % 3)

  issue_lin(0, 0, 0)
  issue_lin(1, 1, 1)
  wait_lin(0, 0, 0)
  issue_gat(0, 0)
  iteration(0, 0, first=True)
  iteration(1, 1)
  nsteady = nch - 4
  n6 = (nsteady // 6) * 6

  def steady(k6, cc):
    for u in range(6):
      iteration(2 + k6 * 6 + u, 2 + u)
    return cc

  lax.fori_loop(0, n6 // 6, steady, 0)
  for u in range(n6, nsteady):
    iteration(2 + u, 2 + u)
  iteration(nch - 2, nch - 2, issue_g=True, issue_l=False)
  iteration(nch - 1, nch - 1, issue_g=False, issue_l=False)
  wait_sca((nch - 1) % 3)


def _splatter(wbuf):
  def _splat(i):
    return plsc.load_gather(wbuf, [jnp.full((16,), i, jnp.int32)])
  return _splat


def _body1(np_, e,
           tab0, srcg, dstg, ino, outo, geom, leng, wpack,
           hpart,
           tab_sp,
           srcb0, srcb1, inob0, inob1, outob0, outob1,
           geomb0, geomb1, lengb0, lengb1,
           dstb0, dstb1, dstb2, mrow0, mrow1, mrow2,
           rows_d0, rows_d1, rows_s0, rows_s1,
           wbuf, slin0, slin1, sgat0, sgat1, ssca0, ssca1, ssca2):
  chunk = CHUNK
  srcb = (srcb0, srcb1); inob = (inob0, inob1); outob = (outob0, outob1)
  geomb = (geomb0, geomb1); lengb = (lengb0, lengb1)
  dstb = (dstb0, dstb1, dstb2); mrow = (mrow0, mrow1, mrow2)
  rows_d = (rows_d0, rows_d1); rows_s = (rows_s0, rows_s1)
  slin = (slin0, slin1); sgat = (sgat0, sgat1); ssca = (ssca0, ssca1, ssca2)

  cid = lax.axis_index("c")
  sid = lax.axis_index("s")
  wid = sid * NC + cid
  npt = np_ // NSUB
  ept = e // (NSUB * NC)
  nch = ept // chunk
  grp = chunk // 16

  iota16 = lax.iota(jnp.int32, 16)
  cols = [jnp.full((16,), c, jnp.int32) for c in range(8)]
  zf = jnp.zeros((16,), jnp.float32)

  done = 0
  while done < npt:
    piece = min(chunk, npt - done)
    rsl = pl.ds(sid * npt + done, piece)
    pltpu.sync_copy(tab0.at[rsl], rows_d0.at[pl.ds(0, piece)])
    pltpu.sync_copy(rows_d0.at[pl.ds(0, piece)], tab_sp.at[rsl])
    done += piece
  pltpu.sync_copy(wpack, wbuf)

  def zero_mrow(j, c):
    ids = j * 16 + iota16
    for b in range(3):
      for col in (0, 1, 2, 6, 7):
        plsc.store_scatter(mrow[b], [ids, cols[col]], zf)
    return c
  lax.fori_loop(0, grp, zero_mrow, 0)

  _splat = _splatter(wbuf)
  plsc.subcore_barrier()

  w1 = [[_splat(1 + k * 3 + j) for j in range(3)] for k in range(6)]
  b1 = [_splat(19 + j) for j in range(3)]

  issue_lin, wait_lin, issue_gat, wait_gat = _pipeline_helpers(
      ept, chunk, wid, tab_sp, srcg, dstg, ino, outo, geom, leng,
      srcb, inob, outob, geomb, lengb, dstb, rows_d, rows_s, slin, sgat)

  def computeA(pb2, pb3):
    rd, rs = rows_d[pb2], rows_s[pb2]

    def grp_body(j, cc):
      ids = j * 16 + iota16
      sl = pl.ds(j * 16, 16)
      xd0 = plsc.load_gather(rd, [ids, cols[0]])
      xd1 = plsc.load_gather(rd, [ids, cols[1]])
      ed = plsc.load_gather(rd, [ids, cols[2]])
      xs0 = plsc.load_gather(rs, [ids, cols[0]])
      xs1 = plsc.load_gather(rs, [ids, cols[1]])
      es = plsc.load_gather(rs, [ids, cols[2]])
      e1 = (es + outob[pb2][sl]) - (ed + inob[pb2][sl])
      wv = 1.0 / (lengb[pb2][sl] + 1.0)
      feats = (xd0, xd1, xs0, xs1, e1, geomb[pb2][sl])
      for jc in range(3):
        acc = b1[jc]
        for k in range(6):
          acc = acc + feats[k] * w1[k][jc]
        acc = jnp.maximum(acc, 0.0) * wv
        plsc.store_scatter(mrow[pb3], [ids, cols[3 + jc]], acc)
      return cc

    lax.fori_loop(0, grp, grp_body, 0)

  def sca_pairA(pb3):
    return (mrow[pb3], tab_sp.at[dstb[pb3]])

  _run_pass(nch, computeA, sca_pairA, ssca,
            issue_lin, wait_lin, issue_gat, wait_gat)
  plsc.subcore_barrier()

  done = 0
  while done < npt:
    piece = min(chunk, npt - done)
    rsl = pl.ds(sid * npt + done, piece)
    osl = pl.ds(cid * np_ + sid * npt + done, piece)
    pltpu.sync_copy(tab_sp.at[rsl], rows_d0.at[pl.ds(0, piece)])
    pltpu.sync_copy(rows_d0.at[pl.ds(0, piece)], hpart.at[osl])
    done += piece


def _body2(np_, e,
           hpart, ntab0, outz, srcg, dstg, ino, outo, geom, leng, wpack,
           opart,
           tab_sp, out_sp,
           srcb0, srcb1, inob0, inob1, outob0, outob1,
           geomb0, geomb1, lengb0, lengb1,
           dstb0, dstb1, dstb2, mbuf0, mbuf1, mbuf2,
           rows_d0, rows_d1, rows_s0, rows_s1,
           wbuf, slin0, slin1, sgat0, sgat1, ssca0, ssca1, ssca2):
  chunk = CHUNK
  srcb = (srcb0, srcb1); inob = (inob0, inob1); outob = (outob0, outob1)
  geomb = (geomb0, geomb1); lengb = (lengb0, lengb1)
  dstb = (dstb0, dstb1, dstb2); mbuf = (mbuf0, mbuf1, mbuf2)
  rows_d = (rows_d0, rows_d1); rows_s = (rows_s0, rows_s1)
  slin = (slin0, slin1); sgat = (sgat0, sgat1); ssca = (ssca0, ssca1, ssca2)

  cid = lax.axis_index("c")
  sid = lax.axis_index("s")
  wid = sid * NC + cid
  npt = np_ // NSUB
  ept = e // (NSUB * NC)
  nch = ept // chunk
  grp = chunk // 16

  iota16 = lax.iota(jnp.int32, 16)
  cols = [jnp.full((16,), c, jnp.int32) for c in range(8)]

  done = 0
  while done < npt:
    piece = min(chunk, npt - done)
    base = sid * npt + done
    rsl = pl.ds(base, piece)
    pltpu.sync_copy(hpart.at[pl.ds(base, piece)], rows_d0.at[pl.ds(0, piece)])
    pltpu.sync_copy(rows_d0.at[pl.ds(0, piece)], tab_sp.at[rsl])
    pltpu.sync_copy(outz.at[pl.ds(base, piece)], mbuf0.at[pl.ds(0, piece)])
    pltpu.sync_copy(mbuf0.at[pl.ds(0, piece)], out_sp.at[pl.ds(base, piece)])

    def mk_ids(j, cc):
      dstb0[pl.ds(j * 16, 16)] = base + j * 16 + iota16
      return cc
    lax.fori_loop(0, piece // 16, mk_ids, 0)
    pltpu.sync_copy(hpart.at[pl.ds(np_ + base, piece)],
                    rows_s0.at[pl.ds(0, piece)])
    pltpu.sync_copy(rows_s0.at[pl.ds(0, piece)],
                    tab_sp.at[dstb0.at[pl.ds(0, piece)]], add=True)
    pltpu.sync_copy(ntab0.at[pl.ds(base, piece)],
                    rows_s0.at[pl.ds(0, piece)])
    pltpu.sync_copy(rows_s0.at[pl.ds(0, piece)],
                    tab_sp.at[dstb0.at[pl.ds(0, piece)]], add=True)
    done += piece
  pltpu.sync_copy(wpack, wbuf)

  _splat = _splatter(wbuf)
  plsc.subcore_barrier()

  w2 = [_splat(22 + k) for k in range(8)]
  b2 = _splat(30)

  issue_lin, wait_lin, issue_gat, wait_gat = _pipeline_helpers(
      ept, chunk, wid, tab_sp, srcg, dstg, ino, outo, geom, leng,
      srcb, inob, outob, geomb, lengb, dstb, rows_d, rows_s, slin, sgat)

  def computeB(pb2, pb3):
    rd, rs = rows_d[pb2], rows_s[pb2]

    def grp_body(j, cc):
      ids = j * 16 + iota16
      sl = pl.ds(j * 16, 16)
      hd0 = plsc.load_gather(rd, [ids, cols[3]])
      hd1 = plsc.load_gather(rd, [ids, cols[4]])
      hd2 = plsc.load_gather(rd, [ids, cols[5]])
      ed = plsc.load_gather(rd, [ids, cols[2]])
      hs0 = plsc.load_gather(rs, [ids, cols[3]])
      hs1 = plsc.load_gather(rs, [ids, cols[4]])
      hs2 = plsc.load_gather(rs, [ids, cols[5]])
      es = plsc.load_gather(rs, [ids, cols[2]])
      e1 = (es + outob[pb2][sl]) - (ed + inob[pb2][sl])
      wv = 1.0 / (lengb[pb2][sl] + 1.0)
      feats = (hd0, hd1, hd2, hs0, hs1, hs2, e1, geomb[pb2][sl])
      acc = b2
      for k in range(8):
        acc = acc + feats[k] * w2[k]
      mbuf[pb3][sl] = jnp.maximum(acc, 0.0) * wv
      return cc

    lax.fori_loop(0, grp, grp_body, 0)

  def sca_pairB(pb3):
    return (mbuf[pb3], out_sp.at[dstb[pb3]])

  _run_pass(nch, computeB, sca_pairB, ssca,
            issue_lin, wait_lin, issue_gat, wait_gat)
  plsc.subcore_barrier()

  done = 0
  while done < npt:
    piece = min(chunk, npt - done)
    rsl = pl.ds(sid * npt + done, piece)
    osl = pl.ds(cid * np_ + sid * npt + done, piece)
    pltpu.sync_copy(out_sp.at[rsl], mbuf0.at[pl.ds(0, piece)])
    pltpu.sync_copy(mbuf0.at[pl.ds(0, piece)], opart.at[osl])
    done += piece


def _body3(np_, opart, out_hbm, b0, b1_, b2_):
  chunk = CHUNK
  sid = lax.axis_index("s")
  npt = np_ // NSUB
  done = 0
  while done < npt:
    piece = min(chunk, npt - done)
    base = sid * npt + done
    pltpu.sync_copy(opart.at[pl.ds(base, piece)], b0.at[pl.ds(0, piece)])
    pltpu.sync_copy(opart.at[pl.ds(np_ + base, piece)],
                    b1_.at[pl.ds(0, piece)])

    def add_grp(j, cc):
      sl = pl.ds(j * 16, 16)
      b2_[sl] = b0[sl] + b1_[sl]
      return cc
    lax.fori_loop(0, piece // 16, add_grp, 0)
    pltpu.sync_copy(b2_.at[pl.ds(0, piece)], out_hbm.at[pl.ds(base, piece)])
    done += piece


@functools.lru_cache(maxsize=None)
def _build(np_, e):
  chunk = CHUNK
  f32, i32 = jnp.float32, jnp.int32
  mesh2 = plsc.VectorSubcoreMesh(
      core_axis_name="c", subcore_axis_name="s", num_cores=NC,
      num_subcores=NSUB)
  mesh1 = plsc.VectorSubcoreMesh(
      core_axis_name="c", subcore_axis_name="s", num_cores=1,
      num_subcores=NSUB)
  cp = pltpu.CompilerParams(
      needs_layout_passes=False, use_tc_tiling_on_sc=False)

  edge_bufs = ([pltpu.VMEM((chunk,), i32) for _ in range(2)] +
               [pltpu.VMEM((chunk,), f32) for _ in range(8)] +
               [pltpu.VMEM((chunk,), i32) for _ in range(3)])
  rows_bufs = [pltpu.VMEM((chunk, 8), f32) for _ in range(4)]
  sems = [pltpu.SemaphoreType.DMA for _ in range(7)]

  k1 = pl.kernel(
      functools.partial(_body1, np_, e),
      out_type=jax.ShapeDtypeStruct((2 * np_, 8), f32),
      mesh=mesh2,
      scratch_types=([pltpu.VMEM_SHARED((np_, 8), f32)] + edge_bufs +
                     [pltpu.VMEM((chunk, 8), f32) for _ in range(3)] +
                     rows_bufs + [pltpu.VMEM((32,), f32)] + sems),
      compiler_params=cp)
  k2 = pl.kernel(
      functools.partial(_body2, np_, e),
      out_type=jax.ShapeDtypeStruct((2 * np_,), f32),
      mesh=mesh2,
      scratch_types=([pltpu.VMEM_SHARED((np_, 8), f32),
                      pltpu.VMEM_SHARED((np_,), f32)] + edge_bufs +
                     [pltpu.VMEM((chunk,), f32) for _ in range(3)] +
                     rows_bufs + [pltpu.VMEM((32,), f32)] + sems),
      compiler_params=cp)
  k3 = pl.kernel(
      functools.partial(_body3, np_),
      out_type=jax.ShapeDtypeStruct((np_,), f32),
      mesh=mesh1,
      scratch_types=[pltpu.VMEM((chunk,), f32) for _ in range(3)],
      compiler_params=cp)

  def run(tab0, ntab0, outz, src, dst, ino, outo, geom, leng, wpack):
    hpart = k1(tab0, src, dst, ino, outo, geom, leng, wpack)
    opart = k2(hpart, ntab0, outz, src, dst, ino, outo, geom, leng, wpack)
    return k3(opart)

  return run


def kernel(x, edge_index, norm_elev, norm_length, norm_geom_1,
           norm_in_offset, norm_out_offset, W1, b1, W2, b2):
  n = x.shape[0]
  e = edge_index.shape[1]
  np_ = ((n + 8 * NSUB - 1) // (8 * NSUB)) * (8 * NSUB)
  padn = np_ - n
  elev = jnp.pad(norm_elev, (0, padn))
  tab0 = jnp.concatenate(
      [jnp.pad(x, ((0, padn), (0, 0))), elev[:, None],
       jnp.zeros((np_, 5), jnp.float32)], axis=1)
  ntab0 = -tab0
  outz = jnp.zeros((np_,), jnp.float32)
  wpack = jnp.concatenate(
      [jnp.zeros((1,), jnp.float32), W1.reshape(-1), b1, W2.reshape(-1), b2,
       jnp.zeros((1,), jnp.float32)])
  out = _build(np_, e)(
      tab0, ntab0, outz, edge_index[0], edge_index[1],
      norm_in_offset, norm_out_offset, norm_geom_1, norm_length, wpack)
  return out[:n, None]
